# baseline (speedup 1.0000x reference)
import functools

import jax
import jax.numpy as jnp
from jax.experimental import pallas as pl
from jax.experimental.pallas import tpu as pltpu

_B, _C, _H, _W = 8, 96, 56, 56
_HW = _H * _W
_HID = _C // 16
_BC = _B * _C


def _perm_const():
    pkey = jax.random.key(42)
    return jax.random.permutation(jax.random.fold_in(pkey, 0), _C)


def _index_body(x_ref, w1_ref, b1_ref, w2_ref, b2_ref, perm_ref, ind_ref):
    s = jnp.mean(x_ref[...], axis=2)
    h = jnp.maximum(
        jax.lax.dot_general(s, w1_ref[...], (((1,), (1,)), ((), ())))
        + b1_ref[...], 0.0)
    lg = jax.lax.dot_general(h, w2_ref[...], (((1,), (1,)), ((), ())))
    sc = jax.nn.sigmoid(lg + b2_ref[...])

    gt = (sc[:, None, :] > sc[:, :, None])
    eq = (sc[:, None, :] == sc[:, :, None])
    ii = jax.lax.broadcasted_iota(jnp.int32, (_B, _C, _C), 1)
    jj = jax.lax.broadcasted_iota(jnp.int32, (_B, _C, _C), 2)
    r = jnp.sum((gt | (eq & (jj < ii))).astype(jnp.int32), axis=2)

    match = (r[:, :, None] == perm_ref[...][0][None, None, :])
    ci = jax.lax.broadcasted_iota(jnp.int32, (_B, _C, _C), 1)
    sg = jnp.sum(jnp.where(match, ci, 0), axis=1)

    base = jax.lax.broadcasted_iota(jnp.int32, (_B, _B, _C), 0) * _C
    ind_ref[...] = base + sg[None, :, :]


def _gather_body(ind_ref, x_ref, o_ref):
    o_ref[...] = x_ref[...]


@jax.jit
def kernel(x, W1, b1, W2, b2):
    x3 = x.reshape(_B, _C, _HW)
    perm = _perm_const().astype(jnp.int32).reshape(1, _C)

    ind = pl.pallas_call(
        _index_body,
        out_shape=jax.ShapeDtypeStruct((_B, _B, _C), jnp.int32),
    )(x3, W1, b1.reshape(1, _HID), W2, b2.reshape(1, _C), perm)

    sg_flat = ind[0].reshape(_BC)

    x4 = x3.reshape(_B, _C, 1, _HW)
    out = pl.pallas_call(
        _gather_body,
        grid_spec=pltpu.PrefetchScalarGridSpec(
            num_scalar_prefetch=1,
            grid=(_BC,),
            in_specs=[
                pl.BlockSpec((_B, 1, 1, _HW), lambda k, sg: (0, sg[k], 0, 0)),
            ],
            out_specs=pl.BlockSpec((_B, 1, 1, _HW), lambda k, sg: (0, k, 0, 0)),
        ),
        out_shape=jax.ShapeDtypeStruct((_B, _BC, 1, _HW), jnp.float32),
    )(sg_flat, x4)

    return out.reshape(_B, _B, _C, _H, _W)


# speedup vs baseline: 1.4447x; 1.4447x over previous
import functools

import jax
import jax.numpy as jnp
from jax.experimental import pallas as pl
from jax.experimental.pallas import tpu as pltpu

_B, _C, _H, _W = 8, 96, 56, 56
_HW = _H * _W
_HID = _C // 16
_BC = _B * _C


def _perm_const():
    pkey = jax.random.key(42)
    return jax.random.permutation(jax.random.fold_in(pkey, 0), _C)


def _index_body(x_ref, w1_ref, b1_ref, w2_ref, b2_ref, perm_ref, ind_ref):
    s = jnp.mean(x_ref[...], axis=(2, 3))
    h = jnp.maximum(
        jax.lax.dot_general(s, w1_ref[...], (((1,), (1,)), ((), ())))
        + b1_ref[...], 0.0)
    lg = jax.lax.dot_general(h, w2_ref[...], (((1,), (1,)), ((), ())))
    sc = jax.nn.sigmoid(lg + b2_ref[...])

    gt = (sc[:, None, :] > sc[:, :, None])
    eq = (sc[:, None, :] == sc[:, :, None])
    ii = jax.lax.broadcasted_iota(jnp.int32, (_B, _C, _C), 1)
    jj = jax.lax.broadcasted_iota(jnp.int32, (_B, _C, _C), 2)
    r = jnp.sum((gt | (eq & (jj < ii))).astype(jnp.int32), axis=2)

    match = (r[:, :, None] == perm_ref[...][0][None, None, :])
    ci = jax.lax.broadcasted_iota(jnp.int32, (_B, _C, _C), 1)
    sg = jnp.sum(jnp.where(match, ci, 0), axis=1)

    base = jax.lax.broadcasted_iota(jnp.int32, (_B, _B, _C), 0) * _C
    ind_ref[...] = base + sg[None, :, :]


def _gather_body(ind_ref, x_ref, o_ref):
    o_ref[...] = x_ref[...]


@jax.jit
def kernel(x, W1, b1, W2, b2):
    perm = _perm_const().astype(jnp.int32).reshape(1, _C)

    ind = pl.pallas_call(
        _index_body,
        out_shape=jax.ShapeDtypeStruct((_B, _B, _C), jnp.int32),
    )(x, W1, b1.reshape(1, _HID), W2, b2.reshape(1, _C), perm)

    sg_flat = ind[0].reshape(_BC)

    out = pl.pallas_call(
        _gather_body,
        grid_spec=pltpu.PrefetchScalarGridSpec(
            num_scalar_prefetch=1,
            grid=(_BC,),
            in_specs=[
                pl.BlockSpec((_B, 1, _H, _W), lambda k, sg: (0, sg[k], 0, 0)),
            ],
            out_specs=pl.BlockSpec((_B, 1, _H, _W), lambda k, sg: (0, k, 0, 0)),
        ),
        out_shape=jax.ShapeDtypeStruct((_B, _BC, _H, _W), jnp.float32),
    )(sg_flat, x)

    return out.reshape(_B, _B, _C, _H, _W)


# speedup vs baseline: 1.5724x; 1.0884x over previous
import functools

import jax
import jax.numpy as jnp
from jax.experimental import pallas as pl
from jax.experimental.pallas import tpu as pltpu

_B, _C, _H, _W = 8, 96, 56, 56
_HW = _H * _W
_HID = _C // 16
_BC = _B * _C


def _perm_const():
    pkey = jax.random.key(42)
    return jax.random.permutation(jax.random.fold_in(pkey, 0), _C)


def _index_body(x_ref, w1_ref, b1_ref, w2_ref, b2_ref, perm_ref, ind_ref):
    s = jnp.mean(x_ref[...], axis=(2, 3))
    h = jnp.maximum(
        jax.lax.dot_general(s, w1_ref[...], (((1,), (1,)), ((), ())))
        + b1_ref[...], 0.0)
    lg = jax.lax.dot_general(h, w2_ref[...], (((1,), (1,)), ((), ())))
    sc = jax.nn.sigmoid(lg + b2_ref[...])

    gt = (sc[:, None, :] > sc[:, :, None])
    eq = (sc[:, None, :] == sc[:, :, None])
    ii = jax.lax.broadcasted_iota(jnp.int32, (_B, _C, _C), 1)
    jj = jax.lax.broadcasted_iota(jnp.int32, (_B, _C, _C), 2)
    r = jnp.sum((gt | (eq & (jj < ii))).astype(jnp.int32), axis=2)

    match = (r[:, :, None] == perm_ref[...][0][None, None, :])
    ci = jax.lax.broadcasted_iota(jnp.int32, (_B, _C, _C), 1)
    sg = jnp.sum(jnp.where(match, ci, 0), axis=1)

    base = jax.lax.broadcasted_iota(jnp.int32, (_B, _B, _C), 0) * _C
    ind_ref[...] = base + sg[None, :, :]


def _gather_body(ind_ref, x_ref, o_ref):
    o_ref[...] = x_ref[...]


@jax.jit
def kernel(x, W1, b1, W2, b2):
    perm = _perm_const().astype(jnp.int32).reshape(1, _C)

    ind = pl.pallas_call(
        _index_body,
        out_shape=jax.ShapeDtypeStruct((_B, _B, _C), jnp.int32),
    )(x, W1, b1.reshape(1, _HID), W2, b2.reshape(1, _C), perm)

    sg_flat = ind[0].reshape(_BC)

    out = pl.pallas_call(
        _gather_body,
        grid_spec=pltpu.PrefetchScalarGridSpec(
            num_scalar_prefetch=1,
            grid=(_BC,),
            in_specs=[
                pl.BlockSpec(
                    (_B, 1, 1, _H, _W), lambda k, sg: (0, sg[k], 0, 0, 0)),
            ],
            out_specs=pl.BlockSpec(
                (_B, 1, 1, _H, _W), lambda k, sg: (0, k // _C, k % _C, 0, 0)),
        ),
        out_shape=jax.ShapeDtypeStruct((_B, _B, _C, _H, _W), jnp.float32),
    )(sg_flat, x[:, :, None])

    return out
